# baseline (speedup 1.0000x reference)
import functools

import jax
import jax.numpy as jnp
from jax.experimental import pallas as pl
from jax.experimental.pallas import tpu as pltpu

_BLK = 2048
_NBUF = 4
_LOOK = 3
_OSEM = 8


def _born_body(x_ref, w1_ref, b1_ref, w2_hbm, b2_ref, out_hbm,
               h_ref, e_ref, mblk_ref, m_ref, s_ref, wbuf_ref, sems, osems,
               *, nb, blk):
    i = pl.program_id(0)

    def _start(block_idx, slot):
        pltpu.make_async_copy(
            w2_hbm.at[pl.ds(block_idx * blk, blk)],
            wbuf_ref.at[slot],
            sems.at[slot],
        ).start()

    def _wait(slot):
        pltpu.make_async_copy(
            w2_hbm.at[pl.ds(0, blk)],
            wbuf_ref.at[slot],
            sems.at[slot],
        ).wait()

    def _out_copy(c):
        return pltpu.make_async_copy(
            e_ref.at[c],
            out_hbm.at[:, pl.ds(c * blk, blk)],
            osems.at[c % _OSEM],
        )

    @pl.when(i == 0)
    def _init():
        for k in range(_LOOK):
            _start(k, k)
        xb = x_ref[...].astype(jnp.bfloat16)
        w1b = w1_ref[...].astype(jnp.bfloat16)
        h = jax.lax.dot_general(xb, w1b, (((1,), (1,)), ((), ())),
                                preferred_element_type=jnp.float32)
        h = jnp.maximum(h + b1_ref[...], 0.0)
        h_ref[...] = h.astype(jnp.bfloat16)
        m_ref[...] = jnp.full(m_ref.shape, -jnp.inf, m_ref.dtype)
        s_ref[...] = jnp.zeros(s_ref.shape, s_ref.dtype)

    @pl.when(i < nb)
    def _logits_block():
        @pl.when(i + _LOOK < nb)
        def _prefetch():
            _start(i + _LOOK, (i + _LOOK) % _NBUF)

        slot = i % _NBUF
        _wait(slot)
        w = wbuf_ref[slot].astype(jnp.bfloat16)
        l = jax.lax.dot_general(h_ref[...], w, (((1,), (1,)), ((), ())),
                                preferred_element_type=jnp.float32)
        l = l + b2_ref[...]
        m_old = m_ref[...]
        m_new = jnp.maximum(m_old, jnp.max(l, axis=1, keepdims=True))
        e = jnp.exp(l - m_new)
        e_ref[i] = e
        mblk_ref[i] = m_new
        alpha = jnp.exp(m_old - m_new)
        s_ref[...] = s_ref[...] * alpha + jnp.sum(e, axis=1, keepdims=True)
        m_ref[...] = m_new

    @pl.when(i == nb)
    def _normalize():
        inv_s = 1.0 / s_ref[...]
        m_fin = m_ref[...]
        for c in range(nb):
            if c >= _OSEM:
                _out_copy(c - _OSEM).wait()
            scale = jnp.exp(mblk_ref[c] - m_fin) * inv_s
            e_ref[c] = e_ref[c] * scale
            _out_copy(c).start()
        for c in range(nb - _OSEM, nb):
            _out_copy(c).wait()


def kernel(x_condition, W1, b1, W2, b2):
    x = x_condition
    if x.ndim == 1:
        x = x[None, :]
    batch, cond = x.shape
    hidden = W1.shape[0]
    n_out = W2.shape[0]
    blk = _BLK
    nb = n_out // blk

    b1_2d = b1.reshape(1, hidden)
    b2_2d = b2.reshape(1, n_out)

    body = functools.partial(_born_body, nb=nb, blk=blk)

    probs = pl.pallas_call(
        body,
        grid=(nb + 1,),
        in_specs=[
            pl.BlockSpec((batch, cond), lambda i: (0, 0)),
            pl.BlockSpec((hidden, cond), lambda i: (0, 0)),
            pl.BlockSpec((1, hidden), lambda i: (0, 0)),
            pl.BlockSpec(memory_space=pl.ANY),
            pl.BlockSpec((1, blk), lambda i: (0, jnp.minimum(i, nb - 1))),
        ],
        out_specs=pl.BlockSpec(memory_space=pl.ANY),
        out_shape=jax.ShapeDtypeStruct((batch, n_out), jnp.float32),
        scratch_shapes=[
            pltpu.VMEM((batch, hidden), jnp.bfloat16),
            pltpu.VMEM((nb, batch, blk), jnp.float32),
            pltpu.VMEM((nb, batch, 1), jnp.float32),
            pltpu.VMEM((batch, 1), jnp.float32),
            pltpu.VMEM((batch, 1), jnp.float32),
            pltpu.VMEM((_NBUF, blk, hidden), jnp.float32),
            pltpu.SemaphoreType.DMA((_NBUF,)),
            pltpu.SemaphoreType.DMA((_OSEM,)),
        ],
        compiler_params=pltpu.CompilerParams(
            dimension_semantics=("arbitrary",),
        ),
    )(x, W1, b1_2d, W2, b2_2d)
    return probs


# speedup vs baseline: 1.0061x; 1.0061x over previous
import functools

import jax
import jax.numpy as jnp
from jax.experimental import pallas as pl
from jax.experimental.pallas import tpu as pltpu

_BLK = 2048
_NBUF = 3
_LOOK = 2
_OSEM = 8
_SUB = 4


def _born_body(x_ref, w1_ref, b1_ref, w2_hbm, b2_ref, out_hbm,
               h_ref, e_ref, mblk_ref, m_ref, s_ref, wbuf_ref, sems, lsems,
               osems, *, nb, blk):
    i = pl.program_id(0)
    sub = blk // _SUB

    def _start(block_idx, slot):
        pltpu.make_async_copy(
            w2_hbm.at[pl.ds(block_idx * blk, blk)],
            wbuf_ref.at[slot],
            sems.at[slot],
        ).start()

    def _wait(slot):
        pltpu.make_async_copy(
            w2_hbm.at[pl.ds(0, blk)],
            wbuf_ref.at[slot],
            sems.at[slot],
        ).wait()

    def _start_last_chunks(slot):
        base = (nb - 1) * blk
        for q in range(_SUB):
            pltpu.make_async_copy(
                w2_hbm.at[pl.ds(base + q * sub, sub)],
                wbuf_ref.at[slot, pl.ds(q * sub, sub)],
                lsems.at[q],
            ).start()

    def _wait_last_chunk(slot, q):
        pltpu.make_async_copy(
            w2_hbm.at[pl.ds(0, sub)],
            wbuf_ref.at[slot, pl.ds(0, sub)],
            lsems.at[q],
        ).wait()

    def _finish_block(l):
        l = l + b2_ref[...]
        m_old = m_ref[...]
        m_new = jnp.maximum(m_old, jnp.max(l, axis=1, keepdims=True))
        e = jnp.exp(l - m_new)
        e_ref[i] = e
        mblk_ref[i] = m_new
        alpha = jnp.exp(m_old - m_new)
        s_ref[...] = s_ref[...] * alpha + jnp.sum(e, axis=1, keepdims=True)
        m_ref[...] = m_new

    def _out_copy(c):
        return pltpu.make_async_copy(
            e_ref.at[c],
            out_hbm.at[:, pl.ds(c * blk, blk)],
            osems.at[c % _OSEM],
        )

    @pl.when(i == 0)
    def _init():
        for k in range(_LOOK):
            _start(k, k)
        xb = x_ref[...].astype(jnp.bfloat16)
        w1b = w1_ref[...].astype(jnp.bfloat16)
        h = jax.lax.dot_general(xb, w1b, (((1,), (1,)), ((), ())),
                                preferred_element_type=jnp.float32)
        h = jnp.maximum(h + b1_ref[...], 0.0)
        h_ref[...] = h.astype(jnp.bfloat16)
        m_ref[...] = jnp.full(m_ref.shape, -jnp.inf, m_ref.dtype)
        s_ref[...] = jnp.zeros(s_ref.shape, s_ref.dtype)

    @pl.when(i < nb - 1)
    def _logits_block():
        @pl.when(i + _LOOK < nb - 1)
        def _prefetch():
            _start(i + _LOOK, (i + _LOOK) % _NBUF)

        @pl.when(i + _LOOK == nb - 1)
        def _prefetch_last():
            _start_last_chunks((nb - 1) % _NBUF)

        slot = i % _NBUF
        _wait(slot)
        w = wbuf_ref[slot].astype(jnp.bfloat16)
        l = jax.lax.dot_general(h_ref[...], w, (((1,), (1,)), ((), ())),
                                preferred_element_type=jnp.float32)
        _finish_block(l)

    @pl.when(i == nb - 1)
    def _logits_last():
        slot = (nb - 1) % _NBUF
        hb = h_ref[...]
        parts = []
        for q in range(_SUB):
            _wait_last_chunk(slot, q)
            w = wbuf_ref[slot, q * sub:(q + 1) * sub, :].astype(jnp.bfloat16)
            parts.append(jax.lax.dot_general(
                hb, w, (((1,), (1,)), ((), ())),
                preferred_element_type=jnp.float32))
        _finish_block(jnp.concatenate(parts, axis=1))

    @pl.when(i == nb)
    def _normalize():
        inv_s = 1.0 / s_ref[...]
        m_fin = m_ref[...]
        for c in range(nb):
            if c >= _OSEM:
                _out_copy(c - _OSEM).wait()
            scale = jnp.exp(mblk_ref[c] - m_fin) * inv_s
            e_ref[c] = e_ref[c] * scale
            _out_copy(c).start()
        for c in range(nb - _OSEM, nb):
            _out_copy(c).wait()


def kernel(x_condition, W1, b1, W2, b2):
    x = x_condition
    if x.ndim == 1:
        x = x[None, :]
    batch, cond = x.shape
    hidden = W1.shape[0]
    n_out = W2.shape[0]
    blk = _BLK
    nb = n_out // blk

    b1_2d = b1.reshape(1, hidden)
    b2_2d = b2.reshape(1, n_out)

    body = functools.partial(_born_body, nb=nb, blk=blk)

    probs = pl.pallas_call(
        body,
        grid=(nb + 1,),
        in_specs=[
            pl.BlockSpec((batch, cond), lambda i: (0, 0)),
            pl.BlockSpec((hidden, cond), lambda i: (0, 0)),
            pl.BlockSpec((1, hidden), lambda i: (0, 0)),
            pl.BlockSpec(memory_space=pl.ANY),
            pl.BlockSpec((1, blk), lambda i: (0, jnp.minimum(i, nb - 1))),
        ],
        out_specs=pl.BlockSpec(memory_space=pl.ANY),
        out_shape=jax.ShapeDtypeStruct((batch, n_out), jnp.float32),
        scratch_shapes=[
            pltpu.VMEM((batch, hidden), jnp.bfloat16),
            pltpu.VMEM((nb, batch, blk), jnp.float32),
            pltpu.VMEM((nb, batch, 1), jnp.float32),
            pltpu.VMEM((batch, 1), jnp.float32),
            pltpu.VMEM((batch, 1), jnp.float32),
            pltpu.VMEM((_NBUF, blk, hidden), jnp.float32),
            pltpu.SemaphoreType.DMA((_NBUF,)),
            pltpu.SemaphoreType.DMA((_SUB,)),
            pltpu.SemaphoreType.DMA((_OSEM,)),
        ],
        compiler_params=pltpu.CompilerParams(
            dimension_semantics=("arbitrary",),
        ),
    )(x, W1, b1_2d, W2, b2_2d)
    return probs


# speedup vs baseline: 1.0100x; 1.0039x over previous
import functools

import jax
import jax.numpy as jnp
from jax.experimental import pallas as pl
from jax.experimental.pallas import tpu as pltpu

_BLK = 2048
_NBUF = 3
_LOOK = 2
_OSEM = 8


def _born_body(x_ref, w1_ref, b1_ref, w2_hbm, b2_ref, out_hbm,
               h_ref, e_ref, mblk_ref, m_ref, s_ref, wbuf_ref, sems, osems,
               *, nb, blk):
    i = pl.program_id(0)

    def _start(block_idx, slot):
        pltpu.make_async_copy(
            w2_hbm.at[pl.ds(block_idx * blk, blk)],
            wbuf_ref.at[slot],
            sems.at[slot],
        ).start()

    def _wait(slot):
        pltpu.make_async_copy(
            w2_hbm.at[pl.ds(0, blk)],
            wbuf_ref.at[slot],
            sems.at[slot],
        ).wait()

    def _out_copy(c):
        return pltpu.make_async_copy(
            e_ref.at[c],
            out_hbm.at[:, pl.ds(c * blk, blk)],
            osems.at[c % _OSEM],
        )

    @pl.when(i == 0)
    def _init():
        for k in range(_LOOK):
            _start(k, k)
        xb = x_ref[...].astype(jnp.bfloat16)
        w1b = w1_ref[...].astype(jnp.bfloat16)
        h = jax.lax.dot_general(xb, w1b, (((1,), (1,)), ((), ())),
                                preferred_element_type=jnp.float32)
        h = jnp.maximum(h + b1_ref[...], 0.0)
        h_ref[...] = h.astype(jnp.bfloat16)
        m_ref[...] = jnp.full(m_ref.shape, -jnp.inf, m_ref.dtype)
        s_ref[...] = jnp.zeros(s_ref.shape, s_ref.dtype)

    @pl.when(i < nb)
    def _logits_block():
        @pl.when(i + _LOOK < nb)
        def _prefetch():
            _start(i + _LOOK, (i + _LOOK) % _NBUF)

        slot = i % _NBUF
        _wait(slot)
        w = wbuf_ref[slot].astype(jnp.bfloat16)
        l = jax.lax.dot_general(h_ref[...], w, (((1,), (1,)), ((), ())),
                                preferred_element_type=jnp.float32)
        l = l + b2_ref[...]
        m_old = m_ref[...]
        m_new = jnp.maximum(m_old, jnp.max(l, axis=1, keepdims=True))
        e = jnp.exp(l - m_new)
        e_ref[i] = e
        mblk_ref[i] = m_new
        alpha = jnp.exp(m_old - m_new)
        s_ref[...] = s_ref[...] * alpha + jnp.sum(e, axis=1, keepdims=True)
        m_ref[...] = m_new

    @pl.when(i == nb)
    def _normalize():
        inv_s = 1.0 / s_ref[...]
        m_fin = m_ref[...]
        for c in range(nb):
            if c >= _OSEM:
                _out_copy(c - _OSEM).wait()
            scale = jnp.exp(mblk_ref[c] - m_fin) * inv_s
            e_ref[c] = e_ref[c] * scale
            _out_copy(c).start()
        for c in range(nb - _OSEM, nb):
            _out_copy(c).wait()


def kernel(x_condition, W1, b1, W2, b2):
    x = x_condition
    if x.ndim == 1:
        x = x[None, :]
    batch, cond = x.shape
    hidden = W1.shape[0]
    n_out = W2.shape[0]
    blk = _BLK
    nb = n_out // blk

    b1_2d = b1.reshape(1, hidden)
    b2_2d = b2.reshape(1, n_out)

    body = functools.partial(_born_body, nb=nb, blk=blk)

    probs = pl.pallas_call(
        body,
        grid=(nb + 1,),
        in_specs=[
            pl.BlockSpec((batch, cond), lambda i: (0, 0)),
            pl.BlockSpec((hidden, cond), lambda i: (0, 0)),
            pl.BlockSpec((1, hidden), lambda i: (0, 0)),
            pl.BlockSpec(memory_space=pl.ANY),
            pl.BlockSpec((1, blk), lambda i: (0, jnp.minimum(i, nb - 1))),
        ],
        out_specs=pl.BlockSpec(memory_space=pl.ANY),
        out_shape=jax.ShapeDtypeStruct((batch, n_out), jnp.float32),
        scratch_shapes=[
            pltpu.VMEM((batch, hidden), jnp.bfloat16),
            pltpu.VMEM((nb, batch, blk), jnp.float32),
            pltpu.VMEM((nb, batch, 1), jnp.float32),
            pltpu.VMEM((batch, 1), jnp.float32),
            pltpu.VMEM((batch, 1), jnp.float32),
            pltpu.VMEM((_NBUF, blk, hidden), jnp.float32),
            pltpu.SemaphoreType.DMA((_NBUF,)),
            pltpu.SemaphoreType.DMA((_OSEM,)),
        ],
        compiler_params=pltpu.CompilerParams(
            dimension_semantics=("arbitrary",),
        ),
    )(x, W1, b1_2d, W2, b2_2d)
    return probs
